# Initial kernel scaffold; baseline (speedup 1.0000x reference)
import jax
import jax.numpy as jnp
from jax.experimental import pallas as pl


def kernel(xyz, features, params):
    raise NotImplementedError("write your pallas kernel here")


# speedup vs baseline: 4.1916x; 4.1916x over previous
import functools

import jax
import jax.numpy as jnp
from jax import lax
from jax.experimental import pallas as pl

_SA_SPECS = [
    dict(npoint=1024, radii=[0.01, 0.03], nsamples=[16, 32]),
    dict(npoint=256, radii=[0.025, 0.05], nsamples=[16, 32]),
    dict(npoint=64, radii=[0.1, 0.15], nsamples=[16, 32]),
]


def _fps_body(npoint, rows_ref, nxyz_ref, nrows_ref):
    Bb = rows_ref.shape[0]
    N = rows_ref.shape[2]
    x = rows_ref[:, 0, :]
    y = rows_ref[:, 1, :]
    z = rows_ref[:, 2, :]
    iota = lax.broadcasted_iota(jnp.int32, (Bb, N), 1)
    iota_p = lax.broadcasted_iota(jnp.int32, (Bb, npoint), 1)

    def body(i, carry):
        dist, cx, cy, cz, nx, ny, nz = carry
        sel = (iota_p == i).astype(jnp.float32)
        nx = nx + sel * cx
        ny = ny + sel * cy
        nz = nz + sel * cz
        d = (x - cx) ** 2 + (y - cy) ** 2 + (z - cz) ** 2
        dist = jnp.minimum(dist, d)
        m = jnp.max(dist, axis=1, keepdims=True)
        sc = jnp.where(dist == m, iota, N)
        j = jnp.min(sc, axis=1, keepdims=True)
        oh = (iota == j).astype(jnp.float32)
        ncx = jnp.sum(x * oh, axis=1, keepdims=True)
        ncy = jnp.sum(y * oh, axis=1, keepdims=True)
        ncz = jnp.sum(z * oh, axis=1, keepdims=True)
        return dist, ncx, ncy, ncz, nx, ny, nz

    dist0 = jnp.full((Bb, N), 1e10, jnp.float32)
    z0 = jnp.zeros((Bb, npoint), jnp.float32)
    _, _, _, _, nx, ny, nz = lax.fori_loop(
        0, npoint, body,
        (dist0, x[:, 0:1], y[:, 0:1], z[:, 0:1], z0, z0, z0))
    nxyz_ref[...] = jnp.concatenate(
        [nx[:, :, None], ny[:, :, None], nz[:, :, None]], axis=2)
    nrows_ref[...] = jnp.concatenate(
        [nx[:, None, :], ny[:, None, :], nz[:, None, :]], axis=1)


def _fps(rows, npoint):
    Bb, _, N = rows.shape
    return pl.pallas_call(
        functools.partial(_fps_body, npoint),
        out_shape=(jax.ShapeDtypeStruct((Bb, npoint, 3), jnp.float32),
                   jax.ShapeDtypeStruct((Bb, 3, npoint), jnp.float32)),
    )(rows)


def _sa_body(radii, nsamples, widths, S_blk, has_feats,
             *refs):
    if has_feats:
        rows_ref, pts_ref, feats_ref, nxyz_ref = refs[:4]
        wrefs = refs[4:-1]
    else:
        rows_ref, pts_ref, nxyz_ref = refs[:3]
        wrefs = refs[3:-1]
    out_ref = refs[-1]
    N = rows_ref.shape[2]
    x = rows_ref[0, 0:1, :]
    y = rows_ref[0, 1:2, :]
    z = rows_ref[0, 2:3, :]
    q = nxyz_ref[0]
    qx, qy, qz = q[:, 0:1], q[:, 1:2], q[:, 2:3]
    d2 = (qx - x) ** 2 + (qy - y) ** 2 + (qz - z) ** 2
    iota = lax.broadcasted_iota(jnp.int32, (1, N), 1)
    P = pts_ref[0]
    F = feats_ref[0] if has_feats else None

    col = 0
    wi = 0
    for scale in range(len(radii)):
        r2 = radii[scale] * radii[scale]
        ns = nsamples[scale]
        W1, b1, W2, b2, W3, b3 = [wrefs[wi + t][...] for t in range(6)]
        wi += 6
        A = jnp.dot(P, W1[:3, :], preferred_element_type=jnp.float32, precision=lax.Precision.HIGHEST) + b1
        if has_feats:
            A = A + jnp.dot(F, W1[3:, :], preferred_element_type=jnp.float32, precision=lax.Precision.HIGHEST)
        qW = jnp.dot(q, W1[:3, :], preferred_element_type=jnp.float32, precision=lax.Precision.HIGHEST)

        score0 = jnp.where(d2 < r2, iota, N)

        def mlp(G):
            h = jnp.maximum(G - qW, 0.0)
            h = jnp.maximum(
                jnp.dot(h, W2, preferred_element_type=jnp.float32, precision=lax.Precision.HIGHEST) + b2, 0.0)
            h = jnp.maximum(
                jnp.dot(h, W3, preferred_element_type=jnp.float32, precision=lax.Precision.HIGHEST) + b3, 0.0)
            return h

        m0 = jnp.min(score0, axis=1, keepdims=True)
        mask0 = score0 == m0
        G0 = jnp.dot(mask0.astype(jnp.float32), A,
                     preferred_element_type=jnp.float32, precision=lax.Precision.HIGHEST)
        G0 = jnp.where(m0 < N, G0, A[0:1, :])
        score1 = jnp.where(mask0, N, score0)
        acc0 = mlp(G0)

        def body(k, carry):
            score, acc = carry
            m = jnp.min(score, axis=1, keepdims=True)
            mask = (score == m) & (m < N)
            G = jnp.dot(mask.astype(jnp.float32), A,
                        preferred_element_type=jnp.float32, precision=lax.Precision.HIGHEST)
            G = jnp.where(m < N, G, G0)
            acc = jnp.maximum(acc, mlp(G))
            score = jnp.where(mask, N, score)
            return score, acc

        _, acc = lax.fori_loop(1, ns, body, (score1, acc0))
        out_ref[0, :, col:col + widths[scale]] = acc
        col += widths[scale]


def _sa_level(rows, pts, feats, nxyz, level_params, radii, nsamples, S_blk):
    Bb, _, N = rows.shape
    S = nxyz.shape[1]
    has_feats = feats is not None
    widths = [lp[-1][0].shape[1] for lp in level_params]
    Ctot = sum(widths)

    in_arrays = [rows, pts] + ([feats] if has_feats else []) + [nxyz]
    in_specs = [
        pl.BlockSpec((1, 3, N), lambda b, s: (b, 0, 0)),
        pl.BlockSpec((1, N, 3), lambda b, s: (b, 0, 0)),
    ]
    if has_feats:
        C = feats.shape[2]
        in_specs.append(pl.BlockSpec((1, N, C), lambda b, s: (b, 0, 0)))
    in_specs.append(pl.BlockSpec((1, S_blk, 3), lambda b, s: (b, s, 0)))
    for lp in level_params:
        for (W, b) in lp:
            in_arrays += [W, b.reshape(1, -1)]
            in_specs += [pl.BlockSpec(W.shape, lambda b_, s_: (0, 0)),
                         pl.BlockSpec((1, b.shape[0]), lambda b_, s_: (0, 0))]

    return pl.pallas_call(
        functools.partial(_sa_body, tuple(radii), tuple(nsamples),
                          tuple(widths), S_blk, has_feats),
        grid=(Bb, S // S_blk),
        in_specs=in_specs,
        out_specs=pl.BlockSpec((1, S_blk, Ctot), lambda b, s: (b, s, 0)),
        out_shape=jax.ShapeDtypeStruct((Bb, S, Ctot), jnp.float32),
    )(*in_arrays)


def _fp_body(U_blk, Cu, *refs):
    if Cu:
        unxyz_ref, krows_ref, fk_ref, fu_ref = refs[:4]
        wrefs = refs[4:-1]
    else:
        unxyz_ref, krows_ref, fk_ref = refs[:3]
        wrefs = refs[3:-1]
    out_ref = refs[-1]
    M = krows_ref.shape[2]
    kx = krows_ref[0, 0:1, :]
    ky = krows_ref[0, 1:2, :]
    kz = krows_ref[0, 2:3, :]
    u = unxyz_ref[0]
    ux, uy, uz = u[:, 0:1], u[:, 1:2], u[:, 2:3]
    d2 = (ux - kx) ** 2 + (uy - ky) ** 2 + (uz - kz) ** 2
    iota = lax.broadcasted_iota(jnp.int32, (1, M), 1)
    fk = fk_ref[0]

    rem = d2
    wsum = None
    vsum = None
    for _ in range(3):
        m = jnp.min(rem, axis=1, keepdims=True)
        sc = jnp.where(rem == m, iota, M)
        j = jnp.min(sc, axis=1, keepdims=True)
        oh = (iota == j)
        row = jnp.dot(oh.astype(jnp.float32), fk,
                      preferred_element_type=jnp.float32, precision=lax.Precision.HIGHEST)
        w = 1.0 / (m + 1e-8)
        wsum = w if wsum is None else wsum + w
        vsum = w * row if vsum is None else vsum + w * row
        rem = jnp.where(oh, jnp.float32(1e30), rem)
    h = vsum / wsum
    if Cu:
        h = jnp.concatenate([h, fu_ref[0]], axis=1)
    W1, b1, W2, b2 = [wrefs[t][...] for t in range(4)]
    h = jnp.maximum(jnp.dot(h, W1, preferred_element_type=jnp.float32, precision=lax.Precision.HIGHEST) + b1,
                    0.0)
    h = jnp.maximum(jnp.dot(h, W2, preferred_element_type=jnp.float32, precision=lax.Precision.HIGHEST) + b2,
                    0.0)
    out_ref[0] = h


def _fp_level(unxyz, krows, fk, fu, layers, U_blk):
    Bb, U, _ = unxyz.shape
    M = krows.shape[2]
    C = fk.shape[2]
    Cu = 0 if fu is None else fu.shape[2]
    Cout = layers[-1][0].shape[1]

    in_arrays = [unxyz, krows, fk] + ([fu] if Cu else [])
    in_specs = [
        pl.BlockSpec((1, U_blk, 3), lambda b, s: (b, s, 0)),
        pl.BlockSpec((1, 3, M), lambda b, s: (b, 0, 0)),
        pl.BlockSpec((1, M, C), lambda b, s: (b, 0, 0)),
    ]
    if Cu:
        in_specs.append(pl.BlockSpec((1, U_blk, Cu), lambda b, s: (b, s, 0)))
    for (W, b) in layers:
        in_arrays += [W, b.reshape(1, -1)]
        in_specs += [pl.BlockSpec(W.shape, lambda b_, s_: (0, 0)),
                     pl.BlockSpec((1, b.shape[0]), lambda b_, s_: (0, 0))]

    return pl.pallas_call(
        functools.partial(_fp_body, U_blk, Cu),
        grid=(Bb, U // U_blk),
        in_specs=in_specs,
        out_specs=pl.BlockSpec((1, U_blk, Cout), lambda b, s: (b, s, 0)),
        out_shape=jax.ShapeDtypeStruct((Bb, U, Cout), jnp.float32),
    )(*in_arrays)


def kernel(xyz, features, params):
    del features
    rows0 = jnp.transpose(xyz, (0, 2, 1))

    l_pts = [xyz]
    l_rows = [rows0]
    l_feats = [None]
    for i, spec in enumerate(_SA_SPECS):
        nxyz, nrows = _fps(l_rows[i], spec['npoint'])
        S_blk = min(spec['npoint'], 512)
        nf = _sa_level(l_rows[i], l_pts[i], l_feats[i], nxyz,
                       params['sa'][i], spec['radii'], spec['nsamples'],
                       S_blk)
        l_pts.append(nxyz)
        l_rows.append(nrows)
        l_feats.append(nf)

    f = l_feats[3]
    for i in (-1, -2, -3):
        fu = l_feats[i - 1] if l_feats[i - 1] is not None else None
        U = l_pts[i - 1].shape[1]
        f = _fp_level(l_pts[i - 1], l_rows[i], f, fu,
                      params['fp'][i], min(U, 1024))

    return f


# speedup vs baseline: 6.5054x; 1.5520x over previous
import functools

import jax
import jax.numpy as jnp
from jax import lax
from jax.experimental import pallas as pl

def _split_gather_dot(mask, table_hi, table_lo):
    mb = mask.astype(jnp.bfloat16)
    return (jnp.dot(mb, table_hi, preferred_element_type=jnp.float32)
            + jnp.dot(mb, table_lo, preferred_element_type=jnp.float32))


def _hi_lo(table):
    hi = table.astype(jnp.bfloat16)
    lo = (table - hi.astype(jnp.float32)).astype(jnp.bfloat16)
    return hi, lo


_SA_SPECS = [
    dict(npoint=1024, radii=[0.01, 0.03], nsamples=[16, 32]),
    dict(npoint=256, radii=[0.025, 0.05], nsamples=[16, 32]),
    dict(npoint=64, radii=[0.1, 0.15], nsamples=[16, 32]),
]


def _fps_body(npoint, rows_ref, nxyz_ref, nrows_ref):
    Bb = rows_ref.shape[0]
    N = rows_ref.shape[2]
    x = rows_ref[:, 0, :]
    y = rows_ref[:, 1, :]
    z = rows_ref[:, 2, :]
    iota = lax.broadcasted_iota(jnp.int32, (Bb, N), 1)
    iota_p = lax.broadcasted_iota(jnp.int32, (Bb, npoint), 1)

    def body(i, carry):
        dist, cx, cy, cz, nx, ny, nz = carry
        sel = (iota_p == i).astype(jnp.float32)
        nx = nx + sel * cx
        ny = ny + sel * cy
        nz = nz + sel * cz
        d = (x - cx) ** 2 + (y - cy) ** 2 + (z - cz) ** 2
        dist = jnp.minimum(dist, d)
        m = jnp.max(dist, axis=1, keepdims=True)
        sc = jnp.where(dist == m, iota, N)
        j = jnp.min(sc, axis=1, keepdims=True)
        oh = (iota == j).astype(jnp.float32)
        ncx = jnp.sum(x * oh, axis=1, keepdims=True)
        ncy = jnp.sum(y * oh, axis=1, keepdims=True)
        ncz = jnp.sum(z * oh, axis=1, keepdims=True)
        return dist, ncx, ncy, ncz, nx, ny, nz

    dist0 = jnp.full((Bb, N), 1e10, jnp.float32)
    z0 = jnp.zeros((Bb, npoint), jnp.float32)
    _, _, _, _, nx, ny, nz = lax.fori_loop(
        0, npoint, body,
        (dist0, x[:, 0:1], y[:, 0:1], z[:, 0:1], z0, z0, z0))
    nxyz_ref[...] = jnp.concatenate(
        [nx[:, :, None], ny[:, :, None], nz[:, :, None]], axis=2)
    nrows_ref[...] = jnp.concatenate(
        [nx[:, None, :], ny[:, None, :], nz[:, None, :]], axis=1)


def _fps(rows, npoint):
    Bb, _, N = rows.shape
    return pl.pallas_call(
        functools.partial(_fps_body, npoint),
        out_shape=(jax.ShapeDtypeStruct((Bb, npoint, 3), jnp.float32),
                   jax.ShapeDtypeStruct((Bb, 3, npoint), jnp.float32)),
    )(rows)


def _sa_body(radii, nsamples, widths, S_blk, has_feats,
             *refs):
    if has_feats:
        rows_ref, pts_ref, feats_ref, nxyz_ref = refs[:4]
        wrefs = refs[4:-1]
    else:
        rows_ref, pts_ref, nxyz_ref = refs[:3]
        wrefs = refs[3:-1]
    out_ref = refs[-1]
    N = rows_ref.shape[2]
    x = rows_ref[0, 0:1, :]
    y = rows_ref[0, 1:2, :]
    z = rows_ref[0, 2:3, :]
    q = nxyz_ref[0]
    qx, qy, qz = q[:, 0:1], q[:, 1:2], q[:, 2:3]
    d2 = (qx - x) ** 2 + (qy - y) ** 2 + (qz - z) ** 2
    iota = lax.broadcasted_iota(jnp.int32, (1, N), 1)
    P = pts_ref[0]
    F = feats_ref[0] if has_feats else None

    col = 0
    wi = 0
    for scale in range(len(radii)):
        r2 = radii[scale] * radii[scale]
        ns = nsamples[scale]
        W1, b1, W2, b2, W3, b3 = [wrefs[wi + t][...] for t in range(6)]
        wi += 6
        A = jnp.dot(P, W1[:3, :], preferred_element_type=jnp.float32, precision=lax.Precision.HIGHEST) + b1
        if has_feats:
            A = A + jnp.dot(F, W1[3:, :], preferred_element_type=jnp.float32, precision=lax.Precision.HIGHEST)
        qW = jnp.dot(q, W1[:3, :], preferred_element_type=jnp.float32, precision=lax.Precision.HIGHEST)
        A_hi, A_lo = _hi_lo(A)

        score0 = jnp.where(d2 < r2, iota, N)

        def mlp(G):
            h = jnp.maximum(G - qW, 0.0)
            h = jnp.maximum(
                jnp.dot(h, W2, preferred_element_type=jnp.float32, precision=lax.Precision.HIGHEST) + b2, 0.0)
            h = jnp.maximum(
                jnp.dot(h, W3, preferred_element_type=jnp.float32, precision=lax.Precision.HIGHEST) + b3, 0.0)
            return h

        m0 = jnp.min(score0, axis=1, keepdims=True)
        mask0 = score0 == m0
        G0 = _split_gather_dot(mask0, A_hi, A_lo)
        G0 = jnp.where(m0 < N, G0, A[0:1, :])
        score1 = jnp.where(mask0, N, score0)
        acc0 = mlp(G0)

        def body(k, carry):
            score, acc = carry
            m = jnp.min(score, axis=1, keepdims=True)
            mask = (score == m) & (m < N)
            G = _split_gather_dot(mask, A_hi, A_lo)
            G = jnp.where(m < N, G, G0)
            acc = jnp.maximum(acc, mlp(G))
            score = jnp.where(mask, N, score)
            return score, acc

        _, acc = lax.fori_loop(1, ns, body, (score1, acc0))
        out_ref[0, :, col:col + widths[scale]] = acc
        col += widths[scale]


def _sa_level(rows, pts, feats, nxyz, level_params, radii, nsamples, S_blk):
    Bb, _, N = rows.shape
    S = nxyz.shape[1]
    has_feats = feats is not None
    widths = [lp[-1][0].shape[1] for lp in level_params]
    Ctot = sum(widths)

    in_arrays = [rows, pts] + ([feats] if has_feats else []) + [nxyz]
    in_specs = [
        pl.BlockSpec((1, 3, N), lambda b, s: (b, 0, 0)),
        pl.BlockSpec((1, N, 3), lambda b, s: (b, 0, 0)),
    ]
    if has_feats:
        C = feats.shape[2]
        in_specs.append(pl.BlockSpec((1, N, C), lambda b, s: (b, 0, 0)))
    in_specs.append(pl.BlockSpec((1, S_blk, 3), lambda b, s: (b, s, 0)))
    for lp in level_params:
        for (W, b) in lp:
            in_arrays += [W, b.reshape(1, -1)]
            in_specs += [pl.BlockSpec(W.shape, lambda b_, s_: (0, 0)),
                         pl.BlockSpec((1, b.shape[0]), lambda b_, s_: (0, 0))]

    return pl.pallas_call(
        functools.partial(_sa_body, tuple(radii), tuple(nsamples),
                          tuple(widths), S_blk, has_feats),
        grid=(Bb, S // S_blk),
        in_specs=in_specs,
        out_specs=pl.BlockSpec((1, S_blk, Ctot), lambda b, s: (b, s, 0)),
        out_shape=jax.ShapeDtypeStruct((Bb, S, Ctot), jnp.float32),
    )(*in_arrays)


def _fp_body(U_blk, Cu, *refs):
    if Cu:
        unxyz_ref, krows_ref, fk_ref, fu_ref = refs[:4]
        wrefs = refs[4:-1]
    else:
        unxyz_ref, krows_ref, fk_ref = refs[:3]
        wrefs = refs[3:-1]
    out_ref = refs[-1]
    M = krows_ref.shape[2]
    kx = krows_ref[0, 0:1, :]
    ky = krows_ref[0, 1:2, :]
    kz = krows_ref[0, 2:3, :]
    u = unxyz_ref[0]
    ux, uy, uz = u[:, 0:1], u[:, 1:2], u[:, 2:3]
    d2 = (ux - kx) ** 2 + (uy - ky) ** 2 + (uz - kz) ** 2
    iota = lax.broadcasted_iota(jnp.int32, (1, M), 1)
    fk = fk_ref[0]
    fk_hi, fk_lo = _hi_lo(fk)

    rem = d2
    wsum = None
    vsum = None
    for _ in range(3):
        m = jnp.min(rem, axis=1, keepdims=True)
        sc = jnp.where(rem == m, iota, M)
        j = jnp.min(sc, axis=1, keepdims=True)
        oh = (iota == j)
        row = _split_gather_dot(oh, fk_hi, fk_lo)
        w = 1.0 / (m + 1e-8)
        wsum = w if wsum is None else wsum + w
        vsum = w * row if vsum is None else vsum + w * row
        rem = jnp.where(oh, jnp.float32(1e30), rem)
    h = vsum / wsum
    if Cu:
        h = jnp.concatenate([h, fu_ref[0]], axis=1)
    W1, b1, W2, b2 = [wrefs[t][...] for t in range(4)]
    h = jnp.maximum(jnp.dot(h, W1, preferred_element_type=jnp.float32, precision=lax.Precision.HIGHEST) + b1,
                    0.0)
    h = jnp.maximum(jnp.dot(h, W2, preferred_element_type=jnp.float32, precision=lax.Precision.HIGHEST) + b2,
                    0.0)
    out_ref[0] = h


def _fp_level(unxyz, krows, fk, fu, layers, U_blk):
    Bb, U, _ = unxyz.shape
    M = krows.shape[2]
    C = fk.shape[2]
    Cu = 0 if fu is None else fu.shape[2]
    Cout = layers[-1][0].shape[1]

    in_arrays = [unxyz, krows, fk] + ([fu] if Cu else [])
    in_specs = [
        pl.BlockSpec((1, U_blk, 3), lambda b, s: (b, s, 0)),
        pl.BlockSpec((1, 3, M), lambda b, s: (b, 0, 0)),
        pl.BlockSpec((1, M, C), lambda b, s: (b, 0, 0)),
    ]
    if Cu:
        in_specs.append(pl.BlockSpec((1, U_blk, Cu), lambda b, s: (b, s, 0)))
    for (W, b) in layers:
        in_arrays += [W, b.reshape(1, -1)]
        in_specs += [pl.BlockSpec(W.shape, lambda b_, s_: (0, 0)),
                     pl.BlockSpec((1, b.shape[0]), lambda b_, s_: (0, 0))]

    return pl.pallas_call(
        functools.partial(_fp_body, U_blk, Cu),
        grid=(Bb, U // U_blk),
        in_specs=in_specs,
        out_specs=pl.BlockSpec((1, U_blk, Cout), lambda b, s: (b, s, 0)),
        out_shape=jax.ShapeDtypeStruct((Bb, U, Cout), jnp.float32),
    )(*in_arrays)


def kernel(xyz, features, params):
    del features
    rows0 = jnp.transpose(xyz, (0, 2, 1))

    l_pts = [xyz]
    l_rows = [rows0]
    l_feats = [None]
    for i, spec in enumerate(_SA_SPECS):
        nxyz, nrows = _fps(l_rows[i], spec['npoint'])
        S_blk = min(spec['npoint'], 512)
        nf = _sa_level(l_rows[i], l_pts[i], l_feats[i], nxyz,
                       params['sa'][i], spec['radii'], spec['nsamples'],
                       S_blk)
        l_pts.append(nxyz)
        l_rows.append(nrows)
        l_feats.append(nf)

    f = l_feats[3]
    for i in (-1, -2, -3):
        fu = l_feats[i - 1] if l_feats[i - 1] is not None else None
        U = l_pts[i - 1].shape[1]
        f = _fp_level(l_pts[i - 1], l_rows[i], f, fu,
                      params['fp'][i], min(U, 1024))

    return f


# speedup vs baseline: 7.9817x; 1.2269x over previous
import functools

import jax
import jax.numpy as jnp
from jax import lax
from jax.experimental import pallas as pl

def _split_gather_dot(mask, table_hi, table_lo):
    mb = mask.astype(jnp.bfloat16)
    return (jnp.dot(mb, table_hi, preferred_element_type=jnp.float32)
            + jnp.dot(mb, table_lo, preferred_element_type=jnp.float32))


def _hi_lo(table):
    hi = table.astype(jnp.bfloat16)
    lo = (table - hi.astype(jnp.float32)).astype(jnp.bfloat16)
    return hi, lo


_SA_SPECS = [
    dict(npoint=1024, radii=[0.01, 0.03], nsamples=[16, 32]),
    dict(npoint=256, radii=[0.025, 0.05], nsamples=[16, 32]),
    dict(npoint=64, radii=[0.1, 0.15], nsamples=[16, 32]),
]


def _fps_body(npoint, rows_ref, nxyz_ref, nrows_ref):
    Bb = rows_ref.shape[0]
    N = rows_ref.shape[2]
    x = rows_ref[:, 0, :]
    y = rows_ref[:, 1, :]
    z = rows_ref[:, 2, :]
    iota = lax.broadcasted_iota(jnp.int32, (Bb, N), 1)
    iota_p = lax.broadcasted_iota(jnp.int32, (Bb, npoint), 1)

    def body(i, carry):
        dist, cx, cy, cz, nx, ny, nz = carry
        sel = (iota_p == i).astype(jnp.float32)
        nx = nx + sel * cx
        ny = ny + sel * cy
        nz = nz + sel * cz
        d = (x - cx) ** 2 + (y - cy) ** 2 + (z - cz) ** 2
        dist = jnp.minimum(dist, d)
        m = jnp.max(dist, axis=1, keepdims=True)
        sc = jnp.where(dist == m, iota, N)
        j = jnp.min(sc, axis=1, keepdims=True)
        oh = (iota == j).astype(jnp.float32)
        ncx = jnp.sum(x * oh, axis=1, keepdims=True)
        ncy = jnp.sum(y * oh, axis=1, keepdims=True)
        ncz = jnp.sum(z * oh, axis=1, keepdims=True)
        return dist, ncx, ncy, ncz, nx, ny, nz

    dist0 = jnp.full((Bb, N), 1e10, jnp.float32)
    z0 = jnp.zeros((Bb, npoint), jnp.float32)
    _, _, _, _, nx, ny, nz = lax.fori_loop(
        0, npoint, body,
        (dist0, x[:, 0:1], y[:, 0:1], z[:, 0:1], z0, z0, z0))
    nxyz_ref[...] = jnp.concatenate(
        [nx[:, :, None], ny[:, :, None], nz[:, :, None]], axis=2)
    nrows_ref[...] = jnp.concatenate(
        [nx[:, None, :], ny[:, None, :], nz[:, None, :]], axis=1)


def _fps(rows, npoint):
    Bb, _, N = rows.shape
    return pl.pallas_call(
        functools.partial(_fps_body, npoint),
        out_shape=(jax.ShapeDtypeStruct((Bb, npoint, 3), jnp.float32),
                   jax.ShapeDtypeStruct((Bb, 3, npoint), jnp.float32)),
    )(rows)


def _table_body(has_feats, *refs):
    if has_feats:
        pts_ref, feats_ref, W_ref, b_ref, out_ref = refs
    else:
        pts_ref, W_ref, b_ref, out_ref = refs
    W = W_ref[...]
    A = jnp.dot(pts_ref[0], W[:3, :], preferred_element_type=jnp.float32,
                precision=lax.Precision.HIGHEST) + b_ref[...]
    if has_feats:
        A = A + jnp.dot(feats_ref[0], W[3:, :],
                        preferred_element_type=jnp.float32,
                        precision=lax.Precision.HIGHEST)
    out_ref[0] = A


def _build_table(pts, feats, W, b):
    Bb, N, _ = pts.shape
    H = W.shape[1]
    has_feats = feats is not None
    in_arrays = [pts] + ([feats] if has_feats else []) + [W, b.reshape(1, -1)]
    in_specs = [pl.BlockSpec((1, N, 3), lambda bi: (bi, 0, 0))]
    if has_feats:
        in_specs.append(
            pl.BlockSpec((1, N, feats.shape[2]), lambda bi: (bi, 0, 0)))
    in_specs += [pl.BlockSpec(W.shape, lambda bi: (0, 0)),
                 pl.BlockSpec((1, H), lambda bi: (0, 0))]
    return pl.pallas_call(
        functools.partial(_table_body, has_feats),
        grid=(Bb,),
        in_specs=in_specs,
        out_specs=pl.BlockSpec((1, N, H), lambda bi: (bi, 0, 0)),
        out_shape=jax.ShapeDtypeStruct((Bb, N, H), jnp.float32),
    )(*in_arrays)


def _sa_body(radii, nsamples, widths, S_blk, sub, N, *refs):
    rows_ref, nxyz_ref = refs[:2]
    a2_refs = refs[2:4]
    wrefs = refs[4:-1]
    out_ref = refs[-1]
    x = rows_ref[0, 0:1, :]
    y = rows_ref[0, 1:2, :]
    z = rows_ref[0, 2:3, :]
    q = nxyz_ref[0]
    qx, qy, qz = q[:, 0:1], q[:, 1:2], q[:, 2:3]
    d2 = (qx - x) ** 2 + (qy - y) ** 2 + (qz - z) ** 2
    iota = lax.broadcasted_iota(jnp.int32, (1, N), 1)
    CH = N // sub
    iota_ch = lax.broadcasted_iota(jnp.int32, (1, CH), 1)

    col = 0
    wi = 0
    for scale in range(len(radii)):
        r2 = radii[scale] * radii[scale]
        ns = nsamples[scale]
        W1, b1, W2, b2, W3, b3 = [wrefs[wi + t][...] for t in range(6)]
        wi += 6
        H = W1.shape[1]
        qW = jnp.dot(q, W1[:3, :], preferred_element_type=jnp.float32,
                     precision=lax.Precision.HIGHEST)
        A2_hi, A2_lo = _hi_lo(a2_refs[scale][0])

        def gather(idx):
            a_s = idx // sub
            oh = iota_ch == a_s
            slab = _split_gather_dot(oh, A2_hi, A2_lo)
            if sub == 1:
                return slab
            b_s = idx - a_s * sub
            G = None
            for bb in range(sub):
                w = (b_s == bb).astype(jnp.float32)
                part = w * slab[:, bb * H:(bb + 1) * H]
                G = part if G is None else G + part
            return G

        def mlp(G):
            h = jnp.maximum(G - qW, 0.0)
            h = jnp.maximum(
                jnp.dot(h, W2, preferred_element_type=jnp.float32,
                        precision=lax.Precision.HIGHEST) + b2, 0.0)
            h = jnp.maximum(
                jnp.dot(h, W3, preferred_element_type=jnp.float32,
                        precision=lax.Precision.HIGHEST) + b3, 0.0)
            return h

        score0 = jnp.where(d2 < r2, iota, N)
        m0 = jnp.min(score0, axis=1, keepdims=True)
        first_idx = jnp.where(m0 < N, m0, 0)
        score1 = jnp.where(score0 == m0, N, score0)
        acc0 = mlp(gather(first_idx))

        def body(k, carry):
            score, acc = carry
            m = jnp.min(score, axis=1, keepdims=True)
            idx = jnp.where(m < N, m, first_idx)
            acc = jnp.maximum(acc, mlp(gather(idx)))
            score = jnp.where(score == m, N, score)
            return score, acc

        _, acc = lax.fori_loop(1, ns, body, (score1, acc0))
        out_ref[0, :, col:col + widths[scale]] = acc
        col += widths[scale]


def _sa_level(rows, nxyz, a2_tables, level_params, radii, nsamples,
              S_blk, sub):
    Bb, _, N = rows.shape
    S = nxyz.shape[1]
    widths = [lp[-1][0].shape[1] for lp in level_params]
    Ctot = sum(widths)

    in_arrays = [rows, nxyz] + list(a2_tables)
    in_specs = [
        pl.BlockSpec((1, 3, N), lambda b, s: (b, 0, 0)),
        pl.BlockSpec((1, S_blk, 3), lambda b, s: (b, s, 0)),
    ]
    for a2 in a2_tables:
        in_specs.append(
            pl.BlockSpec((1,) + a2.shape[1:], lambda b, s: (b, 0, 0)))
    for lp in level_params:
        for (W, b) in lp:
            in_arrays += [W, b.reshape(1, -1)]
            in_specs += [pl.BlockSpec(W.shape, lambda b_, s_: (0, 0)),
                         pl.BlockSpec((1, b.shape[0]), lambda b_, s_: (0, 0))]

    return pl.pallas_call(
        functools.partial(_sa_body, tuple(radii), tuple(nsamples),
                          tuple(widths), S_blk, sub, N),
        grid=(Bb, S // S_blk),
        in_specs=in_specs,
        out_specs=pl.BlockSpec((1, S_blk, Ctot), lambda b, s: (b, s, 0)),
        out_shape=jax.ShapeDtypeStruct((Bb, S, Ctot), jnp.float32),
    )(*in_arrays)


def _fp_body(U_blk, Cu, *refs):
    if Cu:
        unxyz_ref, krows_ref, fk_ref, fu_ref = refs[:4]
        wrefs = refs[4:-1]
    else:
        unxyz_ref, krows_ref, fk_ref = refs[:3]
        wrefs = refs[3:-1]
    out_ref = refs[-1]
    M = krows_ref.shape[2]
    kx = krows_ref[0, 0:1, :]
    ky = krows_ref[0, 1:2, :]
    kz = krows_ref[0, 2:3, :]
    u = unxyz_ref[0]
    ux, uy, uz = u[:, 0:1], u[:, 1:2], u[:, 2:3]
    d2 = (ux - kx) ** 2 + (uy - ky) ** 2 + (uz - kz) ** 2
    iota = lax.broadcasted_iota(jnp.int32, (1, M), 1)
    fk = fk_ref[0]
    fk_hi, fk_lo = _hi_lo(fk)

    rem = d2
    wsum = None
    vsum = None
    for _ in range(3):
        m = jnp.min(rem, axis=1, keepdims=True)
        sc = jnp.where(rem == m, iota, M)
        j = jnp.min(sc, axis=1, keepdims=True)
        oh = (iota == j)
        row = _split_gather_dot(oh, fk_hi, fk_lo)
        w = 1.0 / (m + 1e-8)
        wsum = w if wsum is None else wsum + w
        vsum = w * row if vsum is None else vsum + w * row
        rem = jnp.where(oh, jnp.float32(1e30), rem)
    h = vsum / wsum
    if Cu:
        h = jnp.concatenate([h, fu_ref[0]], axis=1)
    W1, b1, W2, b2 = [wrefs[t][...] for t in range(4)]
    h = jnp.maximum(jnp.dot(h, W1, preferred_element_type=jnp.float32, precision=lax.Precision.HIGHEST) + b1,
                    0.0)
    h = jnp.maximum(jnp.dot(h, W2, preferred_element_type=jnp.float32, precision=lax.Precision.HIGHEST) + b2,
                    0.0)
    out_ref[0] = h


def _fp_level(unxyz, krows, fk, fu, layers, U_blk):
    Bb, U, _ = unxyz.shape
    M = krows.shape[2]
    C = fk.shape[2]
    Cu = 0 if fu is None else fu.shape[2]
    Cout = layers[-1][0].shape[1]

    in_arrays = [unxyz, krows, fk] + ([fu] if Cu else [])
    in_specs = [
        pl.BlockSpec((1, U_blk, 3), lambda b, s: (b, s, 0)),
        pl.BlockSpec((1, 3, M), lambda b, s: (b, 0, 0)),
        pl.BlockSpec((1, M, C), lambda b, s: (b, 0, 0)),
    ]
    if Cu:
        in_specs.append(pl.BlockSpec((1, U_blk, Cu), lambda b, s: (b, s, 0)))
    for (W, b) in layers:
        in_arrays += [W, b.reshape(1, -1)]
        in_specs += [pl.BlockSpec(W.shape, lambda b_, s_: (0, 0)),
                     pl.BlockSpec((1, b.shape[0]), lambda b_, s_: (0, 0))]

    return pl.pallas_call(
        functools.partial(_fp_body, U_blk, Cu),
        grid=(Bb, U // U_blk),
        in_specs=in_specs,
        out_specs=pl.BlockSpec((1, U_blk, Cout), lambda b, s: (b, s, 0)),
        out_shape=jax.ShapeDtypeStruct((Bb, U, Cout), jnp.float32),
    )(*in_arrays)


def kernel(xyz, features, params):
    del features
    rows0 = jnp.transpose(xyz, (0, 2, 1))

    l_pts = [xyz]
    l_rows = [rows0]
    l_feats = [None]
    subs = [16, 8, 1]
    for i, spec in enumerate(_SA_SPECS):
        nxyz, nrows = _fps(l_rows[i], spec['npoint'])
        S_blk = min(spec['npoint'], 512)
        Bb, Np, _ = l_pts[i].shape
        sub = subs[i]
        a2_tables = []
        for (W1, b1), *_rest in params['sa'][i]:
            A = _build_table(l_pts[i], l_feats[i], W1, b1)
            a2_tables.append(A.reshape(Bb, Np // sub, sub * W1.shape[1]))
        nf = _sa_level(l_rows[i], nxyz, a2_tables, params['sa'][i],
                       spec['radii'], spec['nsamples'], S_blk, sub)
        l_pts.append(nxyz)
        l_rows.append(nrows)
        l_feats.append(nf)

    f = l_feats[3]
    for i in (-1, -2, -3):
        fu = l_feats[i - 1] if l_feats[i - 1] is not None else None
        U = l_pts[i - 1].shape[1]
        f = _fp_level(l_pts[i - 1], l_rows[i], f, fu,
                      params['fp'][i], min(U, 1024))

    return f


# speedup vs baseline: 9.6308x; 1.2066x over previous
import functools

import jax
import jax.numpy as jnp
from jax import lax
from jax.experimental import pallas as pl

def _split_gather_dot(mask, table_hi, table_lo):
    mb = mask.astype(jnp.bfloat16)
    return (jnp.dot(mb, table_hi, preferred_element_type=jnp.float32)
            + jnp.dot(mb, table_lo, preferred_element_type=jnp.float32))


def _hi_lo(table):
    hi = table.astype(jnp.bfloat16)
    lo = (table - hi.astype(jnp.float32)).astype(jnp.bfloat16)
    return hi, lo


_SA_SPECS = [
    dict(npoint=1024, radii=[0.01, 0.03], nsamples=[16, 32]),
    dict(npoint=256, radii=[0.025, 0.05], nsamples=[16, 32]),
    dict(npoint=64, radii=[0.1, 0.15], nsamples=[16, 32]),
]


def _fps_body(npoint, rows_ref, nxyz_ref, nrows_ref):
    Bb = rows_ref.shape[0]
    N = rows_ref.shape[2]
    x = rows_ref[:, 0, :]
    y = rows_ref[:, 1, :]
    z = rows_ref[:, 2, :]
    iota = lax.broadcasted_iota(jnp.int32, (Bb, N), 1)
    iota_p = lax.broadcasted_iota(jnp.int32, (Bb, npoint), 1)

    def body(i, carry):
        dist, cx, cy, cz, nx, ny, nz = carry
        sel = (iota_p == i).astype(jnp.float32)
        nx = nx + sel * cx
        ny = ny + sel * cy
        nz = nz + sel * cz
        d = (x - cx) ** 2 + (y - cy) ** 2 + (z - cz) ** 2
        dist = jnp.minimum(dist, d)
        m = jnp.max(dist, axis=1, keepdims=True)
        sc = jnp.where(dist == m, iota, N)
        j = jnp.min(sc, axis=1, keepdims=True)
        oh = (iota == j).astype(jnp.float32)
        ncx = jnp.sum(x * oh, axis=1, keepdims=True)
        ncy = jnp.sum(y * oh, axis=1, keepdims=True)
        ncz = jnp.sum(z * oh, axis=1, keepdims=True)
        return dist, ncx, ncy, ncz, nx, ny, nz

    dist0 = jnp.full((Bb, N), 1e10, jnp.float32)
    z0 = jnp.zeros((Bb, npoint), jnp.float32)
    _, _, _, _, nx, ny, nz = lax.fori_loop(
        0, npoint, body,
        (dist0, x[:, 0:1], y[:, 0:1], z[:, 0:1], z0, z0, z0))
    nxyz_ref[...] = jnp.concatenate(
        [nx[:, :, None], ny[:, :, None], nz[:, :, None]], axis=2)
    nrows_ref[...] = jnp.concatenate(
        [nx[:, None, :], ny[:, None, :], nz[:, None, :]], axis=1)


def _fps(rows, npoint):
    Bb, _, N = rows.shape
    return pl.pallas_call(
        functools.partial(_fps_body, npoint),
        out_shape=(jax.ShapeDtypeStruct((Bb, npoint, 3), jnp.float32),
                   jax.ShapeDtypeStruct((Bb, 3, npoint), jnp.float32)),
    )(rows)


def _table_body(has_feats, *refs):
    if has_feats:
        pts_ref, feats_ref, W_ref, b_ref, out_ref = refs
    else:
        pts_ref, W_ref, b_ref, out_ref = refs
    W = W_ref[...]
    A = jnp.dot(pts_ref[0], W[:3, :], preferred_element_type=jnp.float32,
                precision=lax.Precision.HIGHEST) + b_ref[...]
    if has_feats:
        A = A + jnp.dot(feats_ref[0], W[3:, :],
                        preferred_element_type=jnp.float32,
                        precision=lax.Precision.HIGHEST)
    out_ref[0] = A


def _build_table(pts, feats, W, b):
    Bb, N, _ = pts.shape
    H = W.shape[1]
    has_feats = feats is not None
    in_arrays = [pts] + ([feats] if has_feats else []) + [W, b.reshape(1, -1)]
    in_specs = [pl.BlockSpec((1, N, 3), lambda bi: (bi, 0, 0))]
    if has_feats:
        in_specs.append(
            pl.BlockSpec((1, N, feats.shape[2]), lambda bi: (bi, 0, 0)))
    in_specs += [pl.BlockSpec(W.shape, lambda bi: (0, 0)),
                 pl.BlockSpec((1, H), lambda bi: (0, 0))]
    return pl.pallas_call(
        functools.partial(_table_body, has_feats),
        grid=(Bb,),
        in_specs=in_specs,
        out_specs=pl.BlockSpec((1, N, H), lambda bi: (bi, 0, 0)),
        out_shape=jax.ShapeDtypeStruct((Bb, N, H), jnp.float32),
    )(*in_arrays)


def _sa_body(radii, nsamples, widths, S_blk, sub, N, *refs):
    rows_ref, nxyz_ref = refs[:2]
    a2_refs = refs[2:4]
    wrefs = refs[4:-1]
    out_ref = refs[-1]
    x = rows_ref[0, 0:1, :]
    y = rows_ref[0, 1:2, :]
    z = rows_ref[0, 2:3, :]
    q = nxyz_ref[0]
    qx, qy, qz = q[:, 0:1], q[:, 1:2], q[:, 2:3]
    d2 = (qx - x) ** 2 + (qy - y) ** 2 + (qz - z) ** 2
    iota = lax.broadcasted_iota(jnp.int32, (1, N), 1)
    CH = N // sub
    iota_ch = lax.broadcasted_iota(jnp.int32, (1, CH), 1)

    col = 0
    wi = 0
    for scale in range(len(radii)):
        r2 = radii[scale] * radii[scale]
        ns = nsamples[scale]
        W1, b1, W2, b2, W3, b3 = [wrefs[wi + t][...] for t in range(6)]
        wi += 6
        H = W1.shape[1]
        qW = jnp.dot(q, W1[:3, :], preferred_element_type=jnp.float32,
                     precision=lax.Precision.HIGHEST)
        A2_hi, A2_lo = _hi_lo(a2_refs[scale][0])

        def gather(idx):
            a_s = idx // sub
            oh = iota_ch == a_s
            slab = _split_gather_dot(oh, A2_hi, A2_lo)
            if sub == 1:
                return slab
            b_s = idx - a_s * sub
            G = None
            for bb in range(sub):
                w = (b_s == bb).astype(jnp.float32)
                part = w * slab[:, bb * H:(bb + 1) * H]
                G = part if G is None else G + part
            return G

        def mlp(G):
            h = jnp.maximum(G - qW, 0.0)
            h = jnp.maximum(
                jnp.dot(h, W2, preferred_element_type=jnp.float32,
                        precision=lax.Precision.HIGHEST) + b2, 0.0)
            h = jnp.maximum(
                jnp.dot(h, W3, preferred_element_type=jnp.float32,
                        precision=lax.Precision.HIGHEST) + b3, 0.0)
            return h

        valid = d2 < r2
        m0 = jnp.min(jnp.where(valid, iota, N), axis=1, keepdims=True)
        first_idx = jnp.where(m0 < N, m0, 0)
        acc0 = mlp(gather(first_idx))

        def body(k, carry):
            t, acc = carry
            cand = jnp.where(valid & (iota > t), iota, N)
            m = jnp.min(cand, axis=1, keepdims=True)
            idx = jnp.where(m < N, m, first_idx)
            acc = jnp.maximum(acc, mlp(gather(idx)))
            return m, acc

        _, acc = lax.fori_loop(1, ns, body, (m0, acc0))
        out_ref[0, :, col:col + widths[scale]] = acc
        col += widths[scale]


def _sa_level(rows, nxyz, a2_tables, level_params, radii, nsamples,
              S_blk, sub):
    Bb, _, N = rows.shape
    S = nxyz.shape[1]
    widths = [lp[-1][0].shape[1] for lp in level_params]
    Ctot = sum(widths)

    in_arrays = [rows, nxyz] + list(a2_tables)
    in_specs = [
        pl.BlockSpec((1, 3, N), lambda b, s: (b, 0, 0)),
        pl.BlockSpec((1, S_blk, 3), lambda b, s: (b, s, 0)),
    ]
    for a2 in a2_tables:
        in_specs.append(
            pl.BlockSpec((1,) + a2.shape[1:], lambda b, s: (b, 0, 0)))
    for lp in level_params:
        for (W, b) in lp:
            in_arrays += [W, b.reshape(1, -1)]
            in_specs += [pl.BlockSpec(W.shape, lambda b_, s_: (0, 0)),
                         pl.BlockSpec((1, b.shape[0]), lambda b_, s_: (0, 0))]

    return pl.pallas_call(
        functools.partial(_sa_body, tuple(radii), tuple(nsamples),
                          tuple(widths), S_blk, sub, N),
        grid=(Bb, S // S_blk),
        in_specs=in_specs,
        out_specs=pl.BlockSpec((1, S_blk, Ctot), lambda b, s: (b, s, 0)),
        out_shape=jax.ShapeDtypeStruct((Bb, S, Ctot), jnp.float32),
    )(*in_arrays)


def _fp_body(U_blk, Cu, *refs):
    if Cu:
        unxyz_ref, krows_ref, fk_ref, fu_ref = refs[:4]
        wrefs = refs[4:-1]
    else:
        unxyz_ref, krows_ref, fk_ref = refs[:3]
        wrefs = refs[3:-1]
    out_ref = refs[-1]
    M = krows_ref.shape[2]
    kx = krows_ref[0, 0:1, :]
    ky = krows_ref[0, 1:2, :]
    kz = krows_ref[0, 2:3, :]
    u = unxyz_ref[0]
    ux, uy, uz = u[:, 0:1], u[:, 1:2], u[:, 2:3]
    d2 = (ux - kx) ** 2 + (uy - ky) ** 2 + (uz - kz) ** 2
    iota = lax.broadcasted_iota(jnp.int32, (1, M), 1)
    fk = fk_ref[0]
    fk_hi, fk_lo = _hi_lo(fk)

    rem = d2
    wsum = None
    vsum = None
    for _ in range(3):
        m = jnp.min(rem, axis=1, keepdims=True)
        sc = jnp.where(rem == m, iota, M)
        j = jnp.min(sc, axis=1, keepdims=True)
        oh = (iota == j)
        row = _split_gather_dot(oh, fk_hi, fk_lo)
        w = 1.0 / (m + 1e-8)
        wsum = w if wsum is None else wsum + w
        vsum = w * row if vsum is None else vsum + w * row
        rem = jnp.where(oh, jnp.float32(1e30), rem)
    h = vsum / wsum
    if Cu:
        h = jnp.concatenate([h, fu_ref[0]], axis=1)
    W1, b1, W2, b2 = [wrefs[t][...] for t in range(4)]
    h = jnp.maximum(jnp.dot(h, W1, preferred_element_type=jnp.float32, precision=lax.Precision.HIGHEST) + b1,
                    0.0)
    h = jnp.maximum(jnp.dot(h, W2, preferred_element_type=jnp.float32, precision=lax.Precision.HIGHEST) + b2,
                    0.0)
    out_ref[0] = h


def _fp_level(unxyz, krows, fk, fu, layers, U_blk):
    Bb, U, _ = unxyz.shape
    M = krows.shape[2]
    C = fk.shape[2]
    Cu = 0 if fu is None else fu.shape[2]
    Cout = layers[-1][0].shape[1]

    in_arrays = [unxyz, krows, fk] + ([fu] if Cu else [])
    in_specs = [
        pl.BlockSpec((1, U_blk, 3), lambda b, s: (b, s, 0)),
        pl.BlockSpec((1, 3, M), lambda b, s: (b, 0, 0)),
        pl.BlockSpec((1, M, C), lambda b, s: (b, 0, 0)),
    ]
    if Cu:
        in_specs.append(pl.BlockSpec((1, U_blk, Cu), lambda b, s: (b, s, 0)))
    for (W, b) in layers:
        in_arrays += [W, b.reshape(1, -1)]
        in_specs += [pl.BlockSpec(W.shape, lambda b_, s_: (0, 0)),
                     pl.BlockSpec((1, b.shape[0]), lambda b_, s_: (0, 0))]

    return pl.pallas_call(
        functools.partial(_fp_body, U_blk, Cu),
        grid=(Bb, U // U_blk),
        in_specs=in_specs,
        out_specs=pl.BlockSpec((1, U_blk, Cout), lambda b, s: (b, s, 0)),
        out_shape=jax.ShapeDtypeStruct((Bb, U, Cout), jnp.float32),
    )(*in_arrays)


def kernel(xyz, features, params):
    del features
    rows0 = jnp.transpose(xyz, (0, 2, 1))

    l_pts = [xyz]
    l_rows = [rows0]
    l_feats = [None]
    subs = [16, 8, 1]
    for i, spec in enumerate(_SA_SPECS):
        nxyz, nrows = _fps(l_rows[i], spec['npoint'])
        S_blk = min(spec['npoint'], 512)
        Bb, Np, _ = l_pts[i].shape
        sub = subs[i]
        a2_tables = []
        for (W1, b1), *_rest in params['sa'][i]:
            A = _build_table(l_pts[i], l_feats[i], W1, b1)
            a2_tables.append(A.reshape(Bb, Np // sub, sub * W1.shape[1]))
        nf = _sa_level(l_rows[i], nxyz, a2_tables, params['sa'][i],
                       spec['radii'], spec['nsamples'], S_blk, sub)
        l_pts.append(nxyz)
        l_rows.append(nrows)
        l_feats.append(nf)

    f = l_feats[3]
    for i in (-1, -2, -3):
        fu = l_feats[i - 1] if l_feats[i - 1] is not None else None
        U = l_pts[i - 1].shape[1]
        f = _fp_level(l_pts[i - 1], l_rows[i], f, fu,
                      params['fp'][i], min(U, 1024))

    return f


# speedup vs baseline: 10.7115x; 1.1122x over previous
import functools

import jax
import jax.numpy as jnp
from jax import lax
from jax.experimental import pallas as pl

def _split_gather_dot(mask, table_hi, table_lo):
    mb = mask.astype(jnp.bfloat16)
    return (jnp.dot(mb, table_hi, preferred_element_type=jnp.float32)
            + jnp.dot(mb, table_lo, preferred_element_type=jnp.float32))


def _hi_lo(table):
    hi = table.astype(jnp.bfloat16)
    lo = (table - hi.astype(jnp.float32)).astype(jnp.bfloat16)
    return hi, lo


def _dot3(a, w_hi, w_lo):
    a_hi = a.astype(jnp.bfloat16)
    a_lo = (a - a_hi.astype(jnp.float32)).astype(jnp.bfloat16)
    return (jnp.dot(a_hi, w_hi, preferred_element_type=jnp.float32)
            + jnp.dot(a_lo, w_hi, preferred_element_type=jnp.float32)
            + jnp.dot(a_hi, w_lo, preferred_element_type=jnp.float32))


_SA_SPECS = [
    dict(npoint=1024, radii=[0.01, 0.03], nsamples=[16, 32]),
    dict(npoint=256, radii=[0.025, 0.05], nsamples=[16, 32]),
    dict(npoint=64, radii=[0.1, 0.15], nsamples=[16, 32]),
]


def _fps_body(npoint, rows_ref, nxyz_ref, nrows_ref):
    Bb = rows_ref.shape[0]
    N = rows_ref.shape[2]
    x = rows_ref[:, 0, :]
    y = rows_ref[:, 1, :]
    z = rows_ref[:, 2, :]
    iota = lax.broadcasted_iota(jnp.int32, (Bb, N), 1)
    iota_p = lax.broadcasted_iota(jnp.int32, (Bb, npoint), 1)

    def body(i, carry):
        dist, cx, cy, cz, nx, ny, nz = carry
        sel = (iota_p == i).astype(jnp.float32)
        nx = nx + sel * cx
        ny = ny + sel * cy
        nz = nz + sel * cz
        d = (x - cx) ** 2 + (y - cy) ** 2 + (z - cz) ** 2
        dist = jnp.minimum(dist, d)
        m = jnp.max(dist, axis=1, keepdims=True)
        sc = jnp.where(dist == m, iota, N)
        j = jnp.min(sc, axis=1, keepdims=True)
        oh = (iota == j).astype(jnp.float32)
        ncx = jnp.sum(x * oh, axis=1, keepdims=True)
        ncy = jnp.sum(y * oh, axis=1, keepdims=True)
        ncz = jnp.sum(z * oh, axis=1, keepdims=True)
        return dist, ncx, ncy, ncz, nx, ny, nz

    dist0 = jnp.full((Bb, N), 1e10, jnp.float32)
    z0 = jnp.zeros((Bb, npoint), jnp.float32)
    _, _, _, _, nx, ny, nz = lax.fori_loop(
        0, npoint, body,
        (dist0, x[:, 0:1], y[:, 0:1], z[:, 0:1], z0, z0, z0))
    nxyz_ref[...] = jnp.concatenate(
        [nx[:, :, None], ny[:, :, None], nz[:, :, None]], axis=2)
    nrows_ref[...] = jnp.concatenate(
        [nx[:, None, :], ny[:, None, :], nz[:, None, :]], axis=1)


def _fps(rows, npoint):
    Bb, _, N = rows.shape
    return pl.pallas_call(
        functools.partial(_fps_body, npoint),
        out_shape=(jax.ShapeDtypeStruct((Bb, npoint, 3), jnp.float32),
                   jax.ShapeDtypeStruct((Bb, 3, npoint), jnp.float32)),
    )(rows)


def _table_body(has_feats, *refs):
    if has_feats:
        pts_ref, feats_ref, W_ref, b_ref, out_ref = refs
    else:
        pts_ref, W_ref, b_ref, out_ref = refs
    W = W_ref[...]
    A = jnp.dot(pts_ref[0], W[:3, :], preferred_element_type=jnp.float32,
                precision=lax.Precision.HIGHEST) + b_ref[...]
    if has_feats:
        A = A + jnp.dot(feats_ref[0], W[3:, :],
                        preferred_element_type=jnp.float32,
                        precision=lax.Precision.HIGHEST)
    out_ref[0] = A


def _build_table(pts, feats, W, b):
    Bb, N, _ = pts.shape
    H = W.shape[1]
    has_feats = feats is not None
    in_arrays = [pts] + ([feats] if has_feats else []) + [W, b.reshape(1, -1)]
    in_specs = [pl.BlockSpec((1, N, 3), lambda bi: (bi, 0, 0))]
    if has_feats:
        in_specs.append(
            pl.BlockSpec((1, N, feats.shape[2]), lambda bi: (bi, 0, 0)))
    in_specs += [pl.BlockSpec(W.shape, lambda bi: (0, 0)),
                 pl.BlockSpec((1, H), lambda bi: (0, 0))]
    return pl.pallas_call(
        functools.partial(_table_body, has_feats),
        grid=(Bb,),
        in_specs=in_specs,
        out_specs=pl.BlockSpec((1, N, H), lambda bi: (bi, 0, 0)),
        out_shape=jax.ShapeDtypeStruct((Bb, N, H), jnp.float32),
    )(*in_arrays)


def _sa_body(radii, nsamples, widths, S_blk, sub, N, *refs):
    rows_ref, nxyz_ref = refs[:2]
    a2_refs = refs[2:4]
    wrefs = refs[4:-1]
    out_ref = refs[-1]
    x = rows_ref[0, 0:1, :]
    y = rows_ref[0, 1:2, :]
    z = rows_ref[0, 2:3, :]
    q = nxyz_ref[0]
    qx, qy, qz = q[:, 0:1], q[:, 1:2], q[:, 2:3]
    d2 = (qx - x) ** 2 + (qy - y) ** 2 + (qz - z) ** 2
    iota = lax.broadcasted_iota(jnp.int32, (1, N), 1)
    CH = N // sub
    iota_ch = lax.broadcasted_iota(jnp.int32, (1, CH), 1)

    col = 0
    wi = 0
    for scale in range(len(radii)):
        r2 = radii[scale] * radii[scale]
        ns = nsamples[scale]
        W1, b1, W2, b2, W3, b3 = [wrefs[wi + t][...] for t in range(6)]
        wi += 6
        H = W1.shape[1]
        qW = jnp.dot(q, W1[:3, :], preferred_element_type=jnp.float32,
                     precision=lax.Precision.HIGHEST)
        A2_hi, A2_lo = _hi_lo(a2_refs[scale][0])

        def gather(idx):
            a_s = idx // sub
            oh = iota_ch == a_s
            slab = _split_gather_dot(oh, A2_hi, A2_lo)
            if sub == 1:
                return slab
            b_s = idx - a_s * sub
            G = None
            for bb in range(sub):
                w = (b_s == bb).astype(jnp.float32)
                part = w * slab[:, bb * H:(bb + 1) * H]
                G = part if G is None else G + part
            return G

        W2s, W3s = _hi_lo(W2), _hi_lo(W3)

        def mlp(G):
            h = jnp.maximum(G - qW, 0.0)
            h = jnp.maximum(_dot3(h, *W2s) + b2, 0.0)
            h = jnp.maximum(_dot3(h, *W3s) + b3, 0.0)
            return h

        valid = d2 < r2
        m0 = jnp.min(jnp.where(valid, iota, N), axis=1, keepdims=True)
        first_idx = jnp.where(m0 < N, m0, 0)
        acc0 = mlp(gather(first_idx))

        def body(k, carry):
            t, acc = carry
            cand = jnp.where(valid & (iota > t), iota, N)
            m = jnp.min(cand, axis=1, keepdims=True)
            idx = jnp.where(m < N, m, first_idx)
            acc = jnp.maximum(acc, mlp(gather(idx)))
            return m, acc

        _, acc = lax.fori_loop(1, ns, body, (m0, acc0))
        out_ref[0, :, col:col + widths[scale]] = acc
        col += widths[scale]


def _sa_level(rows, nxyz, a2_tables, level_params, radii, nsamples,
              S_blk, sub):
    Bb, _, N = rows.shape
    S = nxyz.shape[1]
    widths = [lp[-1][0].shape[1] for lp in level_params]
    Ctot = sum(widths)

    in_arrays = [rows, nxyz] + list(a2_tables)
    in_specs = [
        pl.BlockSpec((1, 3, N), lambda b, s: (b, 0, 0)),
        pl.BlockSpec((1, S_blk, 3), lambda b, s: (b, s, 0)),
    ]
    for a2 in a2_tables:
        in_specs.append(
            pl.BlockSpec((1,) + a2.shape[1:], lambda b, s: (b, 0, 0)))
    for lp in level_params:
        for (W, b) in lp:
            in_arrays += [W, b.reshape(1, -1)]
            in_specs += [pl.BlockSpec(W.shape, lambda b_, s_: (0, 0)),
                         pl.BlockSpec((1, b.shape[0]), lambda b_, s_: (0, 0))]

    return pl.pallas_call(
        functools.partial(_sa_body, tuple(radii), tuple(nsamples),
                          tuple(widths), S_blk, sub, N),
        grid=(Bb, S // S_blk),
        in_specs=in_specs,
        out_specs=pl.BlockSpec((1, S_blk, Ctot), lambda b, s: (b, s, 0)),
        out_shape=jax.ShapeDtypeStruct((Bb, S, Ctot), jnp.float32),
    )(*in_arrays)


def _fp_body(U_blk, Cu, *refs):
    if Cu:
        unxyz_ref, krows_ref, fk_ref, fu_ref = refs[:4]
        wrefs = refs[4:-1]
    else:
        unxyz_ref, krows_ref, fk_ref = refs[:3]
        wrefs = refs[3:-1]
    out_ref = refs[-1]
    M = krows_ref.shape[2]
    kx = krows_ref[0, 0:1, :]
    ky = krows_ref[0, 1:2, :]
    kz = krows_ref[0, 2:3, :]
    u = unxyz_ref[0]
    ux, uy, uz = u[:, 0:1], u[:, 1:2], u[:, 2:3]
    d2 = (ux - kx) ** 2 + (uy - ky) ** 2 + (uz - kz) ** 2
    iota = lax.broadcasted_iota(jnp.int32, (1, M), 1)
    fk = fk_ref[0]
    fk_hi, fk_lo = _hi_lo(fk)

    rem = d2
    wsum = None
    vsum = None
    for _ in range(3):
        m = jnp.min(rem, axis=1, keepdims=True)
        sc = jnp.where(rem == m, iota, M)
        j = jnp.min(sc, axis=1, keepdims=True)
        oh = (iota == j)
        row = _split_gather_dot(oh, fk_hi, fk_lo)
        w = 1.0 / (m + 1e-8)
        wsum = w if wsum is None else wsum + w
        vsum = w * row if vsum is None else vsum + w * row
        rem = jnp.where(oh, jnp.float32(1e30), rem)
    h = vsum / wsum
    if Cu:
        h = jnp.concatenate([h, fu_ref[0]], axis=1)
    W1, b1, W2, b2 = [wrefs[t][...] for t in range(4)]
    h = jnp.maximum(_dot3(h, *_hi_lo(W1)) + b1, 0.0)
    h = jnp.maximum(_dot3(h, *_hi_lo(W2)) + b2, 0.0)
    out_ref[0] = h


def _fp_level(unxyz, krows, fk, fu, layers, U_blk):
    Bb, U, _ = unxyz.shape
    M = krows.shape[2]
    C = fk.shape[2]
    Cu = 0 if fu is None else fu.shape[2]
    Cout = layers[-1][0].shape[1]

    in_arrays = [unxyz, krows, fk] + ([fu] if Cu else [])
    in_specs = [
        pl.BlockSpec((1, U_blk, 3), lambda b, s: (b, s, 0)),
        pl.BlockSpec((1, 3, M), lambda b, s: (b, 0, 0)),
        pl.BlockSpec((1, M, C), lambda b, s: (b, 0, 0)),
    ]
    if Cu:
        in_specs.append(pl.BlockSpec((1, U_blk, Cu), lambda b, s: (b, s, 0)))
    for (W, b) in layers:
        in_arrays += [W, b.reshape(1, -1)]
        in_specs += [pl.BlockSpec(W.shape, lambda b_, s_: (0, 0)),
                     pl.BlockSpec((1, b.shape[0]), lambda b_, s_: (0, 0))]

    return pl.pallas_call(
        functools.partial(_fp_body, U_blk, Cu),
        grid=(Bb, U // U_blk),
        in_specs=in_specs,
        out_specs=pl.BlockSpec((1, U_blk, Cout), lambda b, s: (b, s, 0)),
        out_shape=jax.ShapeDtypeStruct((Bb, U, Cout), jnp.float32),
    )(*in_arrays)


def kernel(xyz, features, params):
    del features
    rows0 = jnp.transpose(xyz, (0, 2, 1))

    l_pts = [xyz]
    l_rows = [rows0]
    l_feats = [None]
    subs = [16, 8, 1]
    for i, spec in enumerate(_SA_SPECS):
        nxyz, nrows = _fps(l_rows[i], spec['npoint'])
        S_blk = min(spec['npoint'], 512)
        Bb, Np, _ = l_pts[i].shape
        sub = subs[i]
        a2_tables = []
        for (W1, b1), *_rest in params['sa'][i]:
            A = _build_table(l_pts[i], l_feats[i], W1, b1)
            a2_tables.append(A.reshape(Bb, Np // sub, sub * W1.shape[1]))
        nf = _sa_level(l_rows[i], nxyz, a2_tables, params['sa'][i],
                       spec['radii'], spec['nsamples'], S_blk, sub)
        l_pts.append(nxyz)
        l_rows.append(nrows)
        l_feats.append(nf)

    f = l_feats[3]
    for i in (-1, -2, -3):
        fu = l_feats[i - 1] if l_feats[i - 1] is not None else None
        U = l_pts[i - 1].shape[1]
        f = _fp_level(l_pts[i - 1], l_rows[i], f, fu,
                      params['fp'][i], min(U, 1024))

    return f
